# baseline (speedup 1.0000x reference)
import jax
import jax.numpy as jnp
from jax import lax
from jax.experimental import pallas as pl
from jax.experimental.pallas import tpu as pltpu
from jax.experimental.pallas import tpu_sc as plsc

EMBED_DIM = 64
SEQ = 50
BATCH = 4096
NC, NS, L = 2, 16, 16
NW = NC * NS
BLK = BATCH // NW
GB = BLK // L
ER, ES = EMBED_DIM // 8, 8
NBUF = 3


def _sc_body(tok_hbm, unk_hbm, w_hbm, out_hbm, idx_raw, idx_adj, unk_v,
             rows, trows, gsem, ssem):
    wid = lax.axis_index("s") * NC + lax.axis_index("c")
    pltpu.sync_copy(tok_hbm.at[:, pl.ds(wid * BLK, BLK)], idx_raw)
    pltpu.sync_copy(unk_hbm, unk_v)
    iota16 = lax.iota(jnp.int32, L)

    def adjust(s, _):
        for g in range(GB):
            v = idx_raw[s, pl.ds(g * L, L)]
            idx_adj[s, pl.ds(g * L, L)] = jnp.maximum(v - 2, 0)
        return 0

    def sg(s, b):
        pltpu.async_copy(w_hbm.at[idx_adj.at[s]], rows.at[b], gsem.at[b])

    def wg(s, b):
        pltpu.make_async_copy(w_hbm.at[idx_adj.at[s]], rows.at[b],
                              gsem.at[b]).wait()

    def ss(s, b):
        pltpu.async_copy(trows.at[b], out_hbm.at[s, :, wid], ssem.at[b])

    def ws(s, b):
        pltpu.make_async_copy(trows.at[b], out_hbm.at[s, :, wid],
                              ssem.at[b]).wait()

    def transpose(s, b, with_fix):
        rb = rows.at[b]
        tb = trows.at[b]

        def erow_body(erow, _):
            for esub in range(ES):
                e = erow * ES + esub
                ev = jnp.full((L,), e, jnp.int32)
                if with_fix:
                    unk_e = plsc.load_gather(unk_v, [ev])
                for g in range(GB):
                    bv = g * L + iota16
                    tval = plsc.load_gather(rb, [bv, ev])
                    if with_fix:
                        v = idx_raw[s, pl.ds(g * L, L)]
                        tval = jnp.where(v < 2,
                                         unk_e * v.astype(jnp.float32), tval)
                    tb[erow, esub, pl.ds(g * L, L)] = tval
            return 0

        lax.fori_loop(0, ER, erow_body, 0)

    def step(s, _):
        bg = lax.rem(s, NBUF)
        wg(s, bg)
        lax.cond(s >= NBUF, lambda: ws(s - NBUF, bg), lambda: None)
        bad = jnp.zeros((L,), jnp.bool_)
        for g in range(GB):
            v = idx_raw[s, pl.ds(g * L, L)]
            bad = jnp.logical_or(bad, v < 2)
        nbad = plsc.all_reduce_population_count(bad)
        lax.cond(nbad[0] == 0,
                 lambda: transpose(s, bg, False),
                 lambda: transpose(s, bg, True))
        ss(s, bg)
        lax.cond(s + NBUF < SEQ, lambda: sg(s + NBUF, bg), lambda: None)
        return 0

    lax.fori_loop(0, NBUF, adjust, 0)
    for b in range(NBUF):
        sg(b, b)
    lax.fori_loop(NBUF, SEQ, adjust, 0)

    lax.fori_loop(0, SEQ, step, 0)
    for s in range(SEQ - NBUF, SEQ):
        ws(s, s % NBUF)


@jax.jit
def kernel(tokens, unk, weights):
    tok_t = tokens.T.astype(jnp.int32)
    unk1 = unk.reshape(EMBED_DIM)
    mesh = plsc.VectorSubcoreMesh(core_axis_name="c", subcore_axis_name="s")
    out_lin = pl.kernel(
        _sc_body,
        out_type=jax.ShapeDtypeStruct((SEQ, ER, NW, ES, BLK), jnp.float32),
        mesh=mesh,
        compiler_params=pltpu.CompilerParams(needs_layout_passes=False,
                                             use_tc_tiling_on_sc=False),
        scratch_types=[
            pltpu.VMEM((SEQ, BLK), jnp.int32),
            pltpu.VMEM((SEQ, BLK), jnp.int32),
            pltpu.VMEM((EMBED_DIM,), jnp.float32),
            pltpu.VMEM((NBUF, BLK, EMBED_DIM), jnp.float32),
            pltpu.VMEM((NBUF, ER, ES, BLK), jnp.float32),
            pltpu.SemaphoreType.DMA((NBUF,)),
            pltpu.SemaphoreType.DMA((NBUF,)),
        ],
    )(tok_t, unk1, weights)
    return out_lin.transpose(2, 4, 0, 1, 3).reshape(BATCH, SEQ, EMBED_DIM)
